# Initial kernel scaffold; baseline (speedup 1.0000x reference)
import jax
import jax.numpy as jnp
from jax.experimental import pallas as pl


def kernel(x, edge_index, W):
    raise NotImplementedError("write your pallas kernel here")


# speedup vs baseline: 5.5237x; 5.5237x over previous
import functools

import jax
import jax.numpy as jnp
from jax import lax
from jax.experimental import pallas as pl
from jax.experimental.pallas import tpu as pltpu
from jax.experimental.pallas import tpu_sc as plsc

N_NODES = 10000
N_EDGES = 320000
IN_DIM = 128
HIDDEN_DIM = 256

NUM_CORES = 2
NUM_SUBCORES = 16
NUM_WORKERS = NUM_CORES * NUM_SUBCORES
EDGES_PER_WORKER = N_EDGES // NUM_WORKERS
CHUNK = 80
NUM_CHUNKS = EDGES_PER_WORKER // CHUNK
N_PAD = 10240
ROWS_PER_TILE = N_PAD // NUM_SUBCORES
ZROWS = 32


def _sc_body(x_hbm, src_hbm, dst_hbm, part_hbm,
             src_v, dst_v, rows_v, zbuf_v, acc_sh, sem):
    c = lax.axis_index("c")
    s = lax.axis_index("s")
    wid = c * NUM_SUBCORES + s

    zv = jnp.zeros((16,), jnp.float32)

    def zrow(r, carry):
        def zcol(j, carry2):
            zbuf_v[r, pl.ds(j * 16, 16)] = zv
            return carry2
        return lax.fori_loop(0, IN_DIM // 16, zcol, carry)
    lax.fori_loop(0, ZROWS, zrow, 0)

    row0 = s * ROWS_PER_TILE

    def zacc(k, carry):
        pltpu.sync_copy(zbuf_v, acc_sh.at[pl.ds(row0 + k * ZROWS, ZROWS)])
        return carry
    lax.fori_loop(0, ROWS_PER_TILE // ZROWS, zacc, 0)
    plsc.subcore_barrier()

    ebase = wid * EDGES_PER_WORKER

    def chunk_body(i, carry):
        b = ebase + i * CHUNK
        pltpu.sync_copy(src_hbm.at[pl.ds(b, CHUNK)], src_v)
        pltpu.sync_copy(dst_hbm.at[pl.ds(b, CHUNK)], dst_v)
        pltpu.async_copy(x_hbm.at[src_v], rows_v, sem).wait()
        pltpu.sync_copy(rows_v, acc_sh.at[dst_v], add=True)
        return carry
    lax.fori_loop(0, NUM_CHUNKS, chunk_body, 0)
    plsc.subcore_barrier()

    pltpu.sync_copy(acc_sh.at[pl.ds(row0, ROWS_PER_TILE)],
                    part_hbm.at[c, pl.ds(row0, ROWS_PER_TILE)])


@jax.jit
def _sc_scatter(x, src, dst):
    mesh = plsc.VectorSubcoreMesh(core_axis_name="c", subcore_axis_name="s")
    return pl.kernel(
        _sc_body,
        out_type=jax.ShapeDtypeStruct((NUM_CORES, N_PAD, IN_DIM), jnp.float32),
        mesh=mesh,
        scratch_types=[
            pltpu.VMEM((CHUNK,), jnp.int32),
            pltpu.VMEM((CHUNK,), jnp.int32),
            pltpu.VMEM((CHUNK, IN_DIM), jnp.float32),
            pltpu.VMEM((ZROWS, IN_DIM), jnp.float32),
            pltpu.VMEM_SHARED((N_PAD, IN_DIM), jnp.float32),
            pltpu.SemaphoreType.DMA,
        ],
    )(x, src, dst)


def _mlp_body(x_ref, p_ref, wt_ref, o_ref):
    h = x_ref[...] + p_ref[0] + p_ref[1]
    o_ref[...] = jnp.maximum(
        jnp.dot(h, wt_ref[...], preferred_element_type=jnp.float32), 0.0)


@jax.jit
def _mlp(x, parts, wt):
    blk = 1000
    grid = (N_NODES // blk,)
    return pl.pallas_call(
        _mlp_body,
        grid=grid,
        in_specs=[
            pl.BlockSpec((blk, IN_DIM), lambda i: (i, 0)),
            pl.BlockSpec((NUM_CORES, blk, IN_DIM), lambda i: (0, i, 0)),
            pl.BlockSpec((IN_DIM, HIDDEN_DIM), lambda i: (0, 0)),
        ],
        out_specs=pl.BlockSpec((blk, HIDDEN_DIM), lambda i: (i, 0)),
        out_shape=jax.ShapeDtypeStruct((N_NODES, HIDDEN_DIM), jnp.float32),
    )(x, parts, wt)


def kernel(x, edge_index, W):
    src = edge_index[0].astype(jnp.int32)
    dst = edge_index[1].astype(jnp.int32)
    parts = _sc_scatter(x, src, dst)
    return _mlp(x, parts, W.T)


# speedup vs baseline: 9.1856x; 1.6629x over previous
import functools

import jax
import jax.numpy as jnp
from jax import lax
from jax.experimental import pallas as pl
from jax.experimental.pallas import tpu as pltpu
from jax.experimental.pallas import tpu_sc as plsc

N_NODES = 10000
N_EDGES = 320000
IN_DIM = 128
HIDDEN_DIM = 256

NUM_CORES = 2
NUM_SUBCORES = 16
NUM_WORKERS = NUM_CORES * NUM_SUBCORES
EDGES_PER_WORKER = N_EDGES // NUM_WORKERS
CHUNK = 80
NUM_CHUNKS = EDGES_PER_WORKER // CHUNK
N_PAD = 10240
ROWS_PER_TILE = N_PAD // NUM_SUBCORES
ZROWS = 32


def _sc_body(x_hbm, idx_hbm, part_hbm,
             ia_v, ib_v, rows0_v, rows1_v, acc_sh, gsem0, gsem1, isem0, isem1):
    c = lax.axis_index("c")
    s = lax.axis_index("s")
    wid = c * NUM_SUBCORES + s

    zv = jnp.zeros((16,), jnp.float32)

    def zrow(r, carry):
        def zcol(j, carry2):
            rows0_v[r, pl.ds(j * 16, 16)] = zv
            return carry2
        return lax.fori_loop(0, IN_DIM // 16, zcol, carry)
    lax.fori_loop(0, CHUNK, zrow, 0)

    row0 = s * ROWS_PER_TILE

    def zacc(k, carry):
        pltpu.sync_copy(rows0_v, acc_sh.at[pl.ds(row0 + k * CHUNK, CHUNK)])
        return carry
    lax.fori_loop(0, ROWS_PER_TILE // CHUNK, zacc, 0)

    pltpu.sync_copy(idx_hbm.at[wid, 0], ia_v)
    plsc.subcore_barrier()

    pltpu.async_copy(idx_hbm.at[wid, 1], ib_v, isem1)
    pltpu.async_copy(x_hbm.at[ia_v.at[0]], rows0_v, gsem0)

    def step(a, ia, ib, rx, ry, gsx, gsy, isx, isy):
        pltpu.make_async_copy(idx_hbm.at[wid, a + 1], ib, isy).wait()
        pltpu.make_async_copy(x_hbm.at[ia.at[0]], rx, gsx).wait()
        pltpu.async_copy(x_hbm.at[ib.at[0]], ry, gsy)
        pltpu.sync_copy(rx, acc_sh.at[ia.at[1]], add=True)
        pltpu.async_copy(idx_hbm.at[wid, a + 2], ia, isx)

    def pair_body(i, carry):
        a = 2 * i
        step(a, ia_v, ib_v, rows0_v, rows1_v, gsem0, gsem1, isem0, isem1)
        step(a + 1, ib_v, ia_v, rows1_v, rows0_v, gsem1, gsem0, isem1, isem0)
        return carry
    lax.fori_loop(0, (NUM_CHUNKS - 1) // 2, pair_body, 0)
    pltpu.make_async_copy(idx_hbm.at[wid, NUM_CHUNKS], ib_v, isem1).wait()
    pltpu.make_async_copy(x_hbm.at[ia_v.at[0]], rows0_v, gsem0).wait()
    pltpu.sync_copy(rows0_v, acc_sh.at[ia_v.at[1]], add=True)
    plsc.subcore_barrier()

    pltpu.sync_copy(acc_sh.at[pl.ds(row0, ROWS_PER_TILE)],
                    part_hbm.at[c, pl.ds(row0, ROWS_PER_TILE)])


@jax.jit
def _sc_scatter(x, src, dst):
    mesh = plsc.VectorSubcoreMesh(core_axis_name="c", subcore_axis_name="s")
    blocks = jnp.stack(
        [src.reshape(NUM_WORKERS, NUM_CHUNKS, CHUNK),
         dst.reshape(NUM_WORKERS, NUM_CHUNKS, CHUNK)], axis=2)
    idx = jnp.concatenate([blocks, blocks[:, :1]], axis=1)
    return pl.kernel(
        _sc_body,
        out_type=jax.ShapeDtypeStruct((NUM_CORES, N_PAD, IN_DIM), jnp.float32),
        mesh=mesh,
        scratch_types=[
            pltpu.VMEM((2, CHUNK), jnp.int32),
            pltpu.VMEM((2, CHUNK), jnp.int32),
            pltpu.VMEM((CHUNK, IN_DIM), jnp.float32),
            pltpu.VMEM((CHUNK, IN_DIM), jnp.float32),
            pltpu.VMEM_SHARED((N_PAD, IN_DIM), jnp.float32),
            pltpu.SemaphoreType.DMA,
            pltpu.SemaphoreType.DMA,
            pltpu.SemaphoreType.DMA,
            pltpu.SemaphoreType.DMA,
        ],
    )(x, idx)


def _mlp_body(x_ref, p_ref, wt_ref, o_ref):
    h = x_ref[...] + p_ref[0] + p_ref[1]
    o_ref[...] = jnp.maximum(
        jnp.dot(h, wt_ref[...], preferred_element_type=jnp.float32), 0.0)


@jax.jit
def _mlp(x, parts, wt):
    blk = 1000
    grid = (N_NODES // blk,)
    return pl.pallas_call(
        _mlp_body,
        grid=grid,
        in_specs=[
            pl.BlockSpec((blk, IN_DIM), lambda i: (i, 0)),
            pl.BlockSpec((NUM_CORES, blk, IN_DIM), lambda i: (0, i, 0)),
            pl.BlockSpec((IN_DIM, HIDDEN_DIM), lambda i: (0, 0)),
        ],
        out_specs=pl.BlockSpec((blk, HIDDEN_DIM), lambda i: (i, 0)),
        out_shape=jax.ShapeDtypeStruct((N_NODES, HIDDEN_DIM), jnp.float32),
    )(x, parts, wt)


def kernel(x, edge_index, W):
    src = edge_index[0].astype(jnp.int32)
    dst = edge_index[1].astype(jnp.int32)
    parts = _sc_scatter(x, src, dst)
    return _mlp(x, parts, W.T)
